# baseline (speedup 1.0000x reference)
import functools

import jax
import jax.numpy as jnp
from jax import lax
from jax.experimental import pallas as pl
from jax.experimental.pallas import tpu as pltpu
from jax.experimental.pallas import tpu_sc as plsc

D_MODEL = 64
VOCAB = 1000000
HALF = 524288
NUM_CORES = 2
NUM_SUBCORES = 16
NUM_WORKERS = NUM_CORES * NUM_SUBCORES
CHUNK = 128
NBUF = 4
TBLK = 1024


def _repack_body(wl_ref, wr_ref, out_ref):
    out_ref[:, :D_MODEL] = jnp.transpose(wl_ref[...])
    out_ref[:, D_MODEL:] = jnp.transpose(wr_ref[...])


def _repack_table(weight_t):
    nleft = HALF // TBLK
    last = (VOCAB - 1) // TBLK
    return pl.pallas_call(
        _repack_body,
        grid=(nleft,),
        in_specs=[
            pl.BlockSpec((D_MODEL, TBLK), lambda i: (0, i)),
            pl.BlockSpec(
                (D_MODEL, TBLK), lambda i: (0, jnp.minimum(i + nleft, last))
            ),
        ],
        out_specs=pl.BlockSpec((TBLK, 2 * D_MODEL), lambda i: (i, 0)),
        out_shape=jax.ShapeDtypeStruct((HALF, 2 * D_MODEL), jnp.float32),
    )(weight_t, weight_t)


@functools.partial(jax.jit, static_argnums=(2,))
def _gather_sc(idx, table, chunks_per_w):
    mesh = plsc.VectorSubcoreMesh(core_axis_name="c", subcore_axis_name="s")

    @functools.partial(
        pl.kernel,
        mesh=mesh,
        compiler_params=pltpu.CompilerParams(use_tc_tiling_on_sc=False),
        out_type=jax.ShapeDtypeStruct(
            (NUM_WORKERS, chunks_per_w, CHUNK, D_MODEL), jnp.float32
        ),
        scratch_types=[
            pltpu.VMEM((chunks_per_w, CHUNK), jnp.int32),
            pltpu.VMEM((NBUF, CHUNK, D_MODEL), jnp.float32),
        ]
        + [pltpu.SemaphoreType.DMA] * (2 * NBUF),
    )
    def k(idx_hbm, table_hbm, out_hbm, idx_v, rows_v, *sems):
        gsems = sems[:NBUF]
        ssems = sems[NBUF:]
        wid = lax.axis_index("s") * NUM_CORES + lax.axis_index("c")

        pltpu.sync_copy(idx_hbm.at[wid], idx_v)

        def gather_desc(j, b):
            return pltpu.make_async_copy(
                table_hbm.at[idx_v.at[j]], rows_v.at[b], gsems[b]
            )

        def store_desc(j, b):
            return pltpu.make_async_copy(
                rows_v.at[b], out_hbm.at[wid, j], ssems[b]
            )

        for b in range(NBUF):
            gather_desc(b, b).start()

        def group(g, carry):
            for b in range(NBUF):
                j = g * NBUF + b
                gather_desc(j, b).wait()
                store_desc(j, b).start()
                nxt = j + NBUF

                @pl.when(nxt < chunks_per_w)
                def _():
                    store_desc(j, b).wait()
                    gather_desc(nxt, b).start()

            return carry

        lax.fori_loop(0, chunks_per_w // NBUF, group, 0)

        for b in range(NBUF):
            store_desc(chunks_per_w - NBUF + b, b).wait()

    return k(idx, table)


def kernel(token_ids, weight):
    batch, hist = token_ids.shape
    vocab, d_model = weight.shape
    total = batch * hist
    chunks_per_w = total // (NUM_WORKERS * CHUNK)
    idx = token_ids.reshape(NUM_WORKERS, chunks_per_w, CHUNK)
    out = _gather_sc(idx, weight, chunks_per_w)
    return out.reshape(batch, hist, d_model)


# speedup vs baseline: 1.3874x; 1.3874x over previous
import functools

import jax
import jax.numpy as jnp
from jax import lax
from jax.experimental import pallas as pl
from jax.experimental.pallas import tpu as pltpu
from jax.experimental.pallas import tpu_sc as plsc

D_MODEL = 64
VOCAB = 1000000
HALF = 524288
NUM_CORES = 2
NUM_SUBCORES = 16
NUM_WORKERS = NUM_CORES * NUM_SUBCORES
CHUNK = 128
NBUF = 4
TBLK = 2048


def _repack_body(wl_ref, wr_ref, out_ref):
    eye = jnp.eye(D_MODEL, dtype=jnp.float32)
    dn = (((0,), (0,)), ((), ()))
    out_ref[:, :D_MODEL] = lax.dot_general(
        wl_ref[...], eye, dn, preferred_element_type=jnp.float32)
    out_ref[:, D_MODEL:] = lax.dot_general(
        wr_ref[...], eye, dn, preferred_element_type=jnp.float32)


def _repack_table(weight_t):
    nleft = HALF // TBLK
    last = (VOCAB - 1) // TBLK
    return pl.pallas_call(
        _repack_body,
        grid=(nleft,),
        in_specs=[
            pl.BlockSpec((D_MODEL, TBLK), lambda i: (0, i)),
            pl.BlockSpec(
                (D_MODEL, TBLK), lambda i: (0, jnp.minimum(i + nleft, last))
            ),
        ],
        out_specs=pl.BlockSpec((TBLK, 2 * D_MODEL), lambda i: (i, 0)),
        out_shape=jax.ShapeDtypeStruct((HALF, 2 * D_MODEL), jnp.float32),
    )(weight_t, weight_t)


@functools.partial(jax.jit, static_argnums=(2,))
def _gather_sc(idx, table, chunks_per_w):
    mesh = plsc.VectorSubcoreMesh(core_axis_name="c", subcore_axis_name="s")

    @functools.partial(
        pl.kernel,
        mesh=mesh,
        compiler_params=pltpu.CompilerParams(use_tc_tiling_on_sc=False),
        out_type=jax.ShapeDtypeStruct(
            (NUM_WORKERS, chunks_per_w, CHUNK, D_MODEL), jnp.float32
        ),
        scratch_types=[
            pltpu.VMEM((chunks_per_w, CHUNK), jnp.int32),
            pltpu.VMEM((NBUF, CHUNK, D_MODEL), jnp.float32),
        ]
        + [pltpu.SemaphoreType.DMA] * (2 * NBUF),
    )
    def k(idx_hbm, table_hbm, out_hbm, idx_v, rows_v, *sems):
        gsems = sems[:NBUF]
        ssems = sems[NBUF:]
        wid = lax.axis_index("s") * NUM_CORES + lax.axis_index("c")

        pltpu.sync_copy(idx_hbm.at[wid], idx_v)

        def gather_desc(j, b):
            return pltpu.make_async_copy(
                table_hbm.at[idx_v.at[j]], rows_v.at[b], gsems[b]
            )

        def store_desc(j, b):
            return pltpu.make_async_copy(
                rows_v.at[b], out_hbm.at[wid, j], ssems[b]
            )

        for b in range(NBUF):
            gather_desc(b, b).start()

        def group(g, carry):
            for b in range(NBUF):
                j = g * NBUF + b
                gather_desc(j, b).wait()
                store_desc(j, b).start()
                nxt = j + NBUF

                @pl.when(nxt < chunks_per_w)
                def _():
                    store_desc(j, b).wait()
                    gather_desc(nxt, b).start()

            return carry

        lax.fori_loop(0, chunks_per_w // NBUF, group, 0)

        for b in range(NBUF):
            store_desc(chunks_per_w - NBUF + b, b).wait()

    return k(idx, table)


def kernel(token_ids, weight):
    batch, hist = token_ids.shape
    vocab, d_model = weight.shape
    total = batch * hist
    chunks_per_w = total // (NUM_WORKERS * CHUNK)
    table2 = _repack_table(weight.T)
    table = table2.reshape(2 * HALF, d_model)
    idx = jnp.where(token_ids < HALF, token_ids * 2, (token_ids - HALF) * 2 + 1)
    idx = idx.reshape(NUM_WORKERS, chunks_per_w, CHUNK)
    out = _gather_sc(idx, table, chunks_per_w)
    return out.reshape(batch, hist, d_model)


# speedup vs baseline: 1.5566x; 1.1220x over previous
import functools

import jax
import jax.numpy as jnp
from jax import lax
from jax.experimental import pallas as pl
from jax.experimental.pallas import tpu as pltpu
from jax.experimental.pallas import tpu_sc as plsc

D_MODEL = 64
VOCAB = 1000000
HALF = 524288
NUM_CORES = 2
NUM_SUBCORES = 16
NUM_WORKERS = NUM_CORES * NUM_SUBCORES
CHUNK = 128
NBUF = 4
TBLK = 4096


def _repack_body(wl_ref, wr_ref, out_ref):
    eye = jnp.eye(D_MODEL, dtype=jnp.float32)
    dn = (((0,), (0,)), ((), ()))
    out_ref[:, :D_MODEL] = lax.dot_general(
        wl_ref[...], eye, dn, preferred_element_type=jnp.float32)
    out_ref[:, D_MODEL:] = lax.dot_general(
        wr_ref[...], eye, dn, preferred_element_type=jnp.float32)


def _repack_table(weight_t):
    nleft = HALF // TBLK
    last = (VOCAB - 1) // TBLK
    return pl.pallas_call(
        _repack_body,
        grid=(nleft,),
        in_specs=[
            pl.BlockSpec((D_MODEL, TBLK), lambda i: (0, i)),
            pl.BlockSpec(
                (D_MODEL, TBLK), lambda i: (0, jnp.minimum(i + nleft, last))
            ),
        ],
        out_specs=pl.BlockSpec((TBLK, 2 * D_MODEL), lambda i: (i, 0)),
        out_shape=jax.ShapeDtypeStruct((HALF, 2 * D_MODEL), jnp.float32),
    )(weight_t, weight_t)


@functools.partial(jax.jit, static_argnums=(2,))
def _gather_sc(idx, table, chunks_per_w):
    mesh = plsc.VectorSubcoreMesh(core_axis_name="c", subcore_axis_name="s")

    @functools.partial(
        pl.kernel,
        mesh=mesh,
        compiler_params=pltpu.CompilerParams(use_tc_tiling_on_sc=False),
        out_type=jax.ShapeDtypeStruct(
            (NUM_WORKERS, chunks_per_w, CHUNK, D_MODEL), jnp.float32
        ),
        scratch_types=[
            pltpu.VMEM((chunks_per_w, CHUNK), jnp.int32),
            pltpu.VMEM((NBUF, CHUNK, D_MODEL), jnp.float32),
        ]
        + [pltpu.SemaphoreType.DMA] * (2 * NBUF),
    )
    def k(idx_hbm, table_hbm, out_hbm, idx_v, rows_v, *sems):
        gsems = sems[:NBUF]
        ssems = sems[NBUF:]
        wid = lax.axis_index("s") * NUM_CORES + lax.axis_index("c")

        pltpu.sync_copy(idx_hbm.at[wid], idx_v)

        def gather_desc(j, b):
            return pltpu.make_async_copy(
                table_hbm.at[idx_v.at[j]], rows_v.at[b], gsems[b]
            )

        def store_desc(j, b):
            return pltpu.make_async_copy(
                rows_v.at[b], out_hbm.at[wid, j], ssems[b]
            )

        for b in range(NBUF):
            gather_desc(b, b).start()

        def group(g, carry):
            for b in range(NBUF):
                j = g * NBUF + b
                gather_desc(j, b).wait()
                store_desc(j, b).start()
                nxt = j + NBUF

                @pl.when(nxt < chunks_per_w)
                def _():
                    store_desc(j, b).wait()
                    gather_desc(nxt, b).start()

            return carry

        lax.fori_loop(0, chunks_per_w // NBUF, group, 0)

        for b in range(NBUF):
            store_desc(chunks_per_w - NBUF + b, b).wait()

    return k(idx, table)


def kernel(token_ids, weight):
    batch, hist = token_ids.shape
    vocab, d_model = weight.shape
    total = batch * hist
    chunks_per_w = total // (NUM_WORKERS * CHUNK)
    table2 = _repack_table(weight.T)
    table = table2.reshape(2 * HALF, d_model)
    idx = jnp.where(token_ids < HALF, token_ids * 2, (token_ids - HALF) * 2 + 1)
    idx = idx.reshape(NUM_WORKERS, chunks_per_w, CHUNK)
    out = _gather_sc(idx, table, chunks_per_w)
    return out.reshape(batch, hist, d_model)


# speedup vs baseline: 1.6602x; 1.0665x over previous
import functools

import jax
import jax.numpy as jnp
from jax import lax
from jax.experimental import pallas as pl
from jax.experimental.pallas import tpu as pltpu
from jax.experimental.pallas import tpu_sc as plsc

D_MODEL = 64
VOCAB = 1000000
HALF = 524288
NUM_CORES = 2
NUM_SUBCORES = 16
NUM_WORKERS = NUM_CORES * NUM_SUBCORES
CHUNK = 128
NBUF = 4
TBLK = 8192


def _repack_body(wl_ref, wr_ref, out_ref):
    eye = jnp.eye(D_MODEL, dtype=jnp.float32)
    dn = (((0,), (0,)), ((), ()))
    out_ref[:, :D_MODEL] = lax.dot_general(
        wl_ref[...], eye, dn, preferred_element_type=jnp.float32)
    out_ref[:, D_MODEL:] = lax.dot_general(
        wr_ref[...], eye, dn, preferred_element_type=jnp.float32)


def _repack_table(weight_t):
    nleft = HALF // TBLK
    last = (VOCAB - 1) // TBLK
    return pl.pallas_call(
        _repack_body,
        grid=(nleft,),
        in_specs=[
            pl.BlockSpec((D_MODEL, TBLK), lambda i: (0, i)),
            pl.BlockSpec(
                (D_MODEL, TBLK), lambda i: (0, jnp.minimum(i + nleft, last))
            ),
        ],
        out_specs=pl.BlockSpec((TBLK, 2 * D_MODEL), lambda i: (i, 0)),
        out_shape=jax.ShapeDtypeStruct((HALF, 2 * D_MODEL), jnp.float32),
    )(weight_t, weight_t)


@functools.partial(jax.jit, static_argnums=(2,))
def _gather_sc(idx, table, chunks_per_w):
    mesh = plsc.VectorSubcoreMesh(core_axis_name="c", subcore_axis_name="s")

    @functools.partial(
        pl.kernel,
        mesh=mesh,
        compiler_params=pltpu.CompilerParams(use_tc_tiling_on_sc=False),
        out_type=jax.ShapeDtypeStruct(
            (NUM_WORKERS, chunks_per_w, CHUNK, D_MODEL), jnp.float32
        ),
        scratch_types=[
            pltpu.VMEM((chunks_per_w, CHUNK), jnp.int32),
            pltpu.VMEM((NBUF, CHUNK, D_MODEL), jnp.float32),
        ]
        + [pltpu.SemaphoreType.DMA] * (2 * NBUF),
    )
    def k(idx_hbm, table_hbm, out_hbm, idx_v, rows_v, *sems):
        gsems = sems[:NBUF]
        ssems = sems[NBUF:]
        wid = lax.axis_index("s") * NUM_CORES + lax.axis_index("c")

        pltpu.sync_copy(idx_hbm.at[wid], idx_v)

        def gather_desc(j, b):
            return pltpu.make_async_copy(
                table_hbm.at[idx_v.at[j]], rows_v.at[b], gsems[b]
            )

        def store_desc(j, b):
            return pltpu.make_async_copy(
                rows_v.at[b], out_hbm.at[wid, j], ssems[b]
            )

        for b in range(NBUF):
            gather_desc(b, b).start()

        def group(g, carry):
            for b in range(NBUF):
                j = g * NBUF + b
                gather_desc(j, b).wait()
                store_desc(j, b).start()
                nxt = j + NBUF

                @pl.when(nxt < chunks_per_w)
                def _():
                    store_desc(j, b).wait()
                    gather_desc(nxt, b).start()

            return carry

        lax.fori_loop(0, chunks_per_w // NBUF, group, 0)

        for b in range(NBUF):
            store_desc(chunks_per_w - NBUF + b, b).wait()

    return k(idx, table)


def kernel(token_ids, weight):
    batch, hist = token_ids.shape
    vocab, d_model = weight.shape
    total = batch * hist
    chunks_per_w = total // (NUM_WORKERS * CHUNK)
    table2 = _repack_table(weight.T)
    table = table2.reshape(2 * HALF, d_model)
    idx = jnp.where(token_ids < HALF, token_ids * 2, (token_ids - HALF) * 2 + 1)
    idx = idx.reshape(NUM_WORKERS, chunks_per_w, CHUNK)
    out = _gather_sc(idx, table, chunks_per_w)
    return out.reshape(batch, hist, d_model)


# speedup vs baseline: 1.7003x; 1.0242x over previous
import functools

import jax
import jax.numpy as jnp
from jax import lax
from jax.experimental import pallas as pl
from jax.experimental.pallas import tpu as pltpu
from jax.experimental.pallas import tpu_sc as plsc

D_MODEL = 64
VOCAB = 1000000
HALF = 524288
NUM_CORES = 2
NUM_SUBCORES = 16
NUM_WORKERS = NUM_CORES * NUM_SUBCORES
CHUNK = 128
NBUF = 4
TBLK = 16384


def _repack_body(wl_ref, wr_ref, out_ref):
    eye = jnp.eye(D_MODEL, dtype=jnp.float32)
    dn = (((0,), (0,)), ((), ()))
    out_ref[:, :D_MODEL] = lax.dot_general(
        wl_ref[...], eye, dn, preferred_element_type=jnp.float32)
    out_ref[:, D_MODEL:] = lax.dot_general(
        wr_ref[...], eye, dn, preferred_element_type=jnp.float32)


def _repack_table(weight_t):
    nleft = HALF // TBLK
    last = (VOCAB - 1) // TBLK
    return pl.pallas_call(
        _repack_body,
        grid=(nleft,),
        in_specs=[
            pl.BlockSpec((D_MODEL, TBLK), lambda i: (0, i)),
            pl.BlockSpec(
                (D_MODEL, TBLK), lambda i: (0, jnp.minimum(i + nleft, last))
            ),
        ],
        out_specs=pl.BlockSpec((TBLK, 2 * D_MODEL), lambda i: (i, 0)),
        out_shape=jax.ShapeDtypeStruct((HALF, 2 * D_MODEL), jnp.float32),
    )(weight_t, weight_t)


@functools.partial(jax.jit, static_argnums=(2,))
def _gather_sc(idx, table, chunks_per_w):
    mesh = plsc.VectorSubcoreMesh(core_axis_name="c", subcore_axis_name="s")

    @functools.partial(
        pl.kernel,
        mesh=mesh,
        compiler_params=pltpu.CompilerParams(use_tc_tiling_on_sc=False),
        out_type=jax.ShapeDtypeStruct(
            (NUM_WORKERS, chunks_per_w, CHUNK, D_MODEL), jnp.float32
        ),
        scratch_types=[
            pltpu.VMEM((chunks_per_w, CHUNK), jnp.int32),
            pltpu.VMEM((NBUF, CHUNK, D_MODEL), jnp.float32),
        ]
        + [pltpu.SemaphoreType.DMA] * (2 * NBUF),
    )
    def k(idx_hbm, table_hbm, out_hbm, idx_v, rows_v, *sems):
        gsems = sems[:NBUF]
        ssems = sems[NBUF:]
        wid = lax.axis_index("s") * NUM_CORES + lax.axis_index("c")

        pltpu.sync_copy(idx_hbm.at[wid], idx_v)

        def gather_desc(j, b):
            return pltpu.make_async_copy(
                table_hbm.at[idx_v.at[j]], rows_v.at[b], gsems[b]
            )

        def store_desc(j, b):
            return pltpu.make_async_copy(
                rows_v.at[b], out_hbm.at[wid, j], ssems[b]
            )

        for b in range(NBUF):
            gather_desc(b, b).start()

        def group(g, carry):
            for b in range(NBUF):
                j = g * NBUF + b
                gather_desc(j, b).wait()
                store_desc(j, b).start()
                nxt = j + NBUF

                @pl.when(nxt < chunks_per_w)
                def _():
                    store_desc(j, b).wait()
                    gather_desc(nxt, b).start()

            return carry

        lax.fori_loop(0, chunks_per_w // NBUF, group, 0)

        for b in range(NBUF):
            store_desc(chunks_per_w - NBUF + b, b).wait()

    return k(idx, table)


def kernel(token_ids, weight):
    batch, hist = token_ids.shape
    vocab, d_model = weight.shape
    total = batch * hist
    chunks_per_w = total // (NUM_WORKERS * CHUNK)
    table2 = _repack_table(weight.T)
    table = table2.reshape(2 * HALF, d_model)
    idx = jnp.where(token_ids < HALF, token_ids * 2, (token_ids - HALF) * 2 + 1)
    idx = idx.reshape(NUM_WORKERS, chunks_per_w, CHUNK)
    out = _gather_sc(idx, table, chunks_per_w)
    return out.reshape(batch, hist, d_model)
